# baseline (speedup 1.0000x reference)
import functools

import jax
import jax.numpy as jnp
from jax import lax
from jax.experimental import pallas as pl
from jax.experimental.pallas import tpu as pltpu
from jax.experimental.pallas import tpu_sc as plsc

_N = 10000
_NPAD = 10016
_CH = 128
_NW = 32
_GAMMA = 0.1
_R = 400


def _spmm_body_maker(D, cpws, masks):
  nvec = D // 16

  def body_fn(table, edge_refs, out, idx_v, w_v, rows_v, acc, isems,
              gsem, c, s):
    wid = c * 16 + s

    def zrow(i, carry):
      for k in range(nvec):
        rows_v[i, pl.ds(k * 16, 16)] = jnp.zeros((16,), jnp.float32)
      return carry
    lax.fori_loop(0, _CH, zrow, 0)
    @pl.when(s < 15)
    def _zero():
      for j in range(5):
        pltpu.sync_copy(rows_v, acc.at[pl.ds(s * 640 + j * _CH, _CH)])
    @pl.when(s == 15)
    def _zero15():
      for j in range(3):
        pltpu.sync_copy(rows_v, acc.at[pl.ds(9600 + j * _CH, _CH)])
      pltpu.sync_copy(rows_v.at[pl.ds(0, 32)], acc.at[pl.ds(9984, 32)])
    plsc.subcore_barrier()

    for (idxr, wr), cpw, mask in zip(edge_refs, cpws, masks):
      e0 = wid * cpw
      def group(g, carry):
        pltpu.sync_copy(idxr.at[pl.ds((e0 // 4 + g) * 8, 8)], idx_v)
        for jj in range(4):
          j = g * 4 + jj
          pltpu.sync_copy(wr.at[pl.ds((e0 + j) * _CH, _CH)], w_v)
          g_ = pltpu.make_async_copy(table.at[idx_v.at[jj]], rows_v, gsem)
          g_.start()
          g_.wait()
          def scale(gg, c2):
            wvec = w_v[pl.ds(gg * 16, 16)]
            for ll in range(16):
              wj = jnp.full((16,), wvec[ll], jnp.float32)
              i = gg * 16 + ll
              for k in range(nvec):
                if k in mask:
                  rows_v[i, pl.ds(k * 16, 16)] = (
                      rows_v[i, pl.ds(k * 16, 16)] * wj)
                else:
                  rows_v[i, pl.ds(k * 16, 16)] = jnp.zeros((16,), jnp.float32)
            return c2
          lax.fori_loop(0, _CH // 16, scale, 0)
          pltpu.sync_copy(rows_v, acc.at[idx_v.at[4 + jj]], add=True)
        return carry
      lax.fori_loop(0, cpw // 4, group, 0)

    plsc.subcore_barrier()
    r0 = s * 624
    for j in range(4):
      pltpu.sync_copy(acc.at[pl.ds(r0 + j * _CH, _CH)],
                      out.at[c, pl.ds(r0 + j * _CH, _CH)])
    pltpu.sync_copy(acc.at[pl.ds(r0 + 512, 112)],
                    out.at[c, pl.ds(r0 + 512, 112)])
    @pl.when(s == 15)
    def _tail():
      pltpu.sync_copy(acc.at[pl.ds(9984, 16)], out.at[c, pl.ds(9984, 16)])

  return body_fn


def _spmm_scratch(D):
  return [
      pltpu.VMEM((8, _CH), jnp.int32),
      pltpu.VMEM((_CH,), jnp.float32),
      pltpu.VMEM((_CH, D), jnp.float32),
      pltpu.VMEM_SHARED((_NPAD, D), jnp.float32),
      pltpu.SemaphoreType.DMA,
      pltpu.SemaphoreType.DMA,
      pltpu.SemaphoreType.DMA,
  ]


@functools.lru_cache(maxsize=None)
def _mesh():
  return plsc.VectorSubcoreMesh(core_axis_name="c", subcore_axis_name="s")


@functools.lru_cache(maxsize=None)
def _make_spmm(D, Epad):
  cpw = Epad // (_NW * _CH)
  body_fn = _spmm_body_maker(D, (cpw,), (tuple(range(D // 16)),))

  @functools.partial(
      pl.kernel,
      out_type=jax.ShapeDtypeStruct((2, _N, D), jnp.float32),
      mesh=_mesh(),
      scratch_types=_spmm_scratch(D),
  )
  def spmm(table, idx, w, out, idx_v, w_v, rows_v, acc,
           isem0, isem1, gsem):
    c = lax.axis_index("c")
    s = lax.axis_index("s")
    body_fn(table, ((idx, w),), out, idx_v, w_v, rows_v, acc,
            (isem0, isem1), gsem, c, s)

  return spmm


@functools.lru_cache(maxsize=None)
def _make_spmm_pair(D, Epad1, Epad2, halfvecs):
  cpw1 = Epad1 // (_NW * _CH)
  cpw2 = Epad2 // (_NW * _CH)
  body_fn = _spmm_body_maker(
      D, (cpw1, cpw2),
      (tuple(range(halfvecs)), tuple(range(halfvecs, 2 * halfvecs))))

  @functools.partial(
      pl.kernel,
      out_type=jax.ShapeDtypeStruct((2, _N, D), jnp.float32),
      mesh=_mesh(),
      scratch_types=_spmm_scratch(D),
  )
  def spmm(table, idx1, w1, idx2, w2, out,
           idx_v, w_v, rows_v, acc, isem0, isem1, gsem):
    c = lax.axis_index("c")
    s = lax.axis_index("s")
    body_fn(table, ((idx1, w1), (idx2, w2)), out,
            idx_v, w_v, rows_v, acc, (isem0, isem1), gsem, c, s)

  return spmm


def _mm2(x, Wa, Wb):
  def body(x_ref, wa_ref, wb_ref, oa_ref, ob_ref):
    xb = x_ref[...]
    oa_ref[...] = jnp.dot(xb, wa_ref[...], preferred_element_type=jnp.float32)
    ob_ref[...] = jnp.dot(xb, wb_ref[...], preferred_element_type=jnp.float32)
  return pl.pallas_call(
      body,
      grid=(_N // _R,),
      in_specs=[
          pl.BlockSpec((_R, 128), lambda i: (i, 0)),
          pl.BlockSpec((128, 128), lambda i: (0, 0)),
          pl.BlockSpec((128, 128), lambda i: (0, 0)),
      ],
      out_specs=[pl.BlockSpec((_R, 128), lambda i: (i, 0))] * 2,
      out_shape=[jax.ShapeDtypeStruct((_N, 128), jnp.float32)] * 2,
  )(x, Wa, Wb)


def _combine1(x, xw1, xw2, p_ei1, p_kf1, p_ei2, p_ks2, b11, b21,
              Gx, bx, Gh1, bh1, Gh2, bh2, W12p, W22p):
  def body(x_ref, xw1_ref, xw2_ref, pe1_ref, pk1_ref, pe2_ref, pk2_ref,
           b11_ref, b21_ref, gx_ref, bx_ref, gh1_ref, bh1_ref, gh2_ref,
           bh2_ref, w12_ref, w22_ref, hw_ref, gates_ref):
    xb = x_ref[...]
    g = jnp.dot(xb, gx_ref[...], preferred_element_type=jnp.float32) + bx_ref[...]
    s1 = jax.nn.sigmoid(g[:, 0:1])
    dk1 = g[:, 1:2]
    s2 = jax.nn.sigmoid(g[:, 2:3])
    dk2 = g[:, 3:4]
    a1 = pe1_ref[0] + pe1_ref[1] + b11_ref[...]
    k1 = pk1_ref[0] + pk1_ref[1] + b11_ref[...]
    i1 = xw1_ref[...] + b11_ref[...]
    h1 = jnp.maximum(s1 * a1 + (1.0 - s1) * k1 + _GAMMA * dk1 * i1, 0.0)
    a2 = pe2_ref[0] + pe2_ref[1] + b21_ref[...]
    k2 = pk2_ref[0] + pk2_ref[1] + b21_ref[...]
    i2 = xw2_ref[...] + b21_ref[...]
    h2 = jnp.maximum(s2 * a2 + (1.0 - s2) * k2 + _GAMMA * dk2 * i2, 0.0)
    hw_ref[...] = (
        jnp.dot(h1, w12_ref[...], preferred_element_type=jnp.float32)
        + jnp.dot(h2, w22_ref[...], preferred_element_type=jnp.float32))
    g1 = jnp.dot(h1, gh1_ref[...], preferred_element_type=jnp.float32) + bh1_ref[...]
    g2 = jnp.dot(h2, gh2_ref[...], preferred_element_type=jnp.float32) + bh2_ref[...]
    gates_ref[...] = jnp.concatenate([
        jax.nn.sigmoid(g1[:, 0:1]), g1[:, 1:2], jax.nn.sigmoid(g1[:, 2:3]),
        jax.nn.sigmoid(g2[:, 0:1]), g2[:, 1:2], jax.nn.sigmoid(g2[:, 2:3]),
        jnp.zeros_like(g1[:, 0:2]),
    ], axis=1)

  part = lambda: pl.BlockSpec((2, _R, 128), lambda i: (0, i, 0))
  return pl.pallas_call(
      body,
      grid=(_N // _R,),
      in_specs=[
          pl.BlockSpec((_R, 128), lambda i: (i, 0)),
          pl.BlockSpec((_R, 128), lambda i: (i, 0)),
          pl.BlockSpec((_R, 128), lambda i: (i, 0)),
          part(), part(), part(), part(),
          pl.BlockSpec((1, 128), lambda i: (0, 0)),
          pl.BlockSpec((1, 128), lambda i: (0, 0)),
          pl.BlockSpec((128, 4), lambda i: (0, 0)),
          pl.BlockSpec((1, 4), lambda i: (0, 0)),
          pl.BlockSpec((128, 3), lambda i: (0, 0)),
          pl.BlockSpec((1, 3), lambda i: (0, 0)),
          pl.BlockSpec((128, 3), lambda i: (0, 0)),
          pl.BlockSpec((1, 3), lambda i: (0, 0)),
          pl.BlockSpec((128, 128), lambda i: (0, 0)),
          pl.BlockSpec((128, 128), lambda i: (0, 0)),
      ],
      out_specs=[
          pl.BlockSpec((_R, 128), lambda i: (i, 0)),
          pl.BlockSpec((_R, 8), lambda i: (i, 0)),
      ],
      out_shape=[
          jax.ShapeDtypeStruct((_N, 128), jnp.float32),
          jax.ShapeDtypeStruct((_N, 8), jnp.float32),
      ],
  )(x, xw1, xw2, p_ei1, p_kf1, p_ei2, p_ks2, b11, b21,
    Gx, bx, Gh1, bh1, Gh2, bh2, W12p, W22p)


def _final(hw, gates, q_a, q_k, b12p, b22p):
  def body(hw_ref, g_ref, qa_ref, qk_ref, b12_ref, b22_ref, o_ref):
    g = g_ref[...]
    s21 = g[:, 0:1]
    dk21 = g[:, 1:2]
    w1 = g[:, 2:3]
    s22 = g[:, 3:4]
    dk22 = g[:, 4:5]
    w2 = g[:, 5:6]
    qa = qa_ref[0] + qa_ref[1]
    qk = qk_ref[0] + qk_ref[1]
    hw = hw_ref[...]
    a1 = qa[:, 0:48] + b12_ref[...]
    k1 = qk[:, 0:48] + b12_ref[...]
    i1 = hw[:, 0:48] + b12_ref[...]
    o1 = s21 * a1 + (1.0 - s21) * k1 + _GAMMA * dk21 * i1
    a2 = qa[:, 64:112] + b22_ref[...]
    k2 = qk[:, 64:112] + b22_ref[...]
    i2 = hw[:, 64:112] + b22_ref[...]
    o2 = s22 * a2 + (1.0 - s22) * k2 + _GAMMA * dk22 * i2
    out = w1 * o1 + w2 * o2
    ids = lax.broadcasted_iota(jnp.int32, out.shape, 1)
    valid = ids < 40
    m = jnp.max(jnp.where(valid, out, -jnp.inf), axis=1, keepdims=True)
    e = jnp.where(valid, jnp.exp(out - m), 0.0)
    se = jnp.sum(e, axis=1, keepdims=True)
    o_ref[...] = out - m - jnp.log(se)

  part = lambda: pl.BlockSpec((2, _R, 128), lambda i: (0, i, 0))
  return pl.pallas_call(
      body,
      grid=(_N // _R,),
      in_specs=[
          pl.BlockSpec((_R, 128), lambda i: (i, 0)),
          pl.BlockSpec((_R, 8), lambda i: (i, 0)),
          part(), part(),
          pl.BlockSpec((1, 48), lambda i: (0, 0)),
          pl.BlockSpec((1, 48), lambda i: (0, 0)),
      ],
      out_specs=pl.BlockSpec((_R, 48), lambda i: (i, 0)),
      out_shape=jax.ShapeDtypeStruct((_N, 48), jnp.float32),
  )(hw, gates, q_a, q_k, b12p, b22p)


def _pad_edges(ei, ew, epad):
  e = ew.shape[0]
  pad = epad - e
  src = jnp.concatenate([ei[0], jnp.zeros((pad,), jnp.int32)])
  dst = jnp.concatenate([ei[1], jnp.zeros((pad,), jnp.int32)])
  w = jnp.concatenate([ew, jnp.zeros((pad,), jnp.float32)])
  idx = jnp.concatenate([src.reshape(-1, 4, _CH), dst.reshape(-1, 4, _CH)],
                        axis=1).reshape(-1, _CH)
  return idx, w


def kernel(x, edge_index, edge_weight, kf_edge_index, kf_edge_weight,
           ks_edge_index, ks_edge_weight, W11, b11, W12, b12, W21, b21,
           W22, b22, scores1_0, scores1_1, scores2_0, scores2_1, bias1_0,
           bias2_0, Dk1_0, Dk1_1, Dk2_0, Dk2_1, Dbias1_0, Dbias1_1,
           Dbias2_0, Dbias2_1, ec1, eb1, ec2, eb2):
  grain = _NW * _CH * 4
  epad = ((edge_weight.shape[0] + grain - 1) // grain) * grain
  ekpad = ((kf_edge_weight.shape[0] + grain - 1) // grain) * grain
  idx_e, w_e = _pad_edges(edge_index, edge_weight, epad)
  idx_f, w_f = _pad_edges(kf_edge_index, kf_edge_weight, ekpad)
  idx_s, w_s = _pad_edges(ks_edge_index, ks_edge_weight, ekpad)

  xw1, xw2 = _mm2(x, W11, W21)

  spmm_e = _make_spmm(128, epad)
  spmm_k = _make_spmm(128, ekpad)
  p_ei1 = spmm_e(xw1, idx_e, w_e)
  p_kf1 = spmm_k(xw1, idx_f, w_f)
  p_ei2 = spmm_e(xw2, idx_e, w_e)
  p_ks2 = spmm_k(xw2, idx_s, w_s)

  Gx = jnp.concatenate([scores1_0, Dk1_0, scores2_0, Dk2_0], axis=1)
  bx = jnp.stack([bias1_0[0], Dbias1_0[0], bias2_0[0], Dbias2_0[0]]).reshape(1, 4)
  Gh1 = jnp.concatenate([scores1_1, Dk1_1, ec1], axis=1)
  bh1 = jnp.stack([bias1_0[0], Dbias1_1[0], eb1[0]]).reshape(1, 3)
  Gh2 = jnp.concatenate([scores2_1, Dk2_1, ec2], axis=1)
  bh2 = jnp.stack([bias2_0[0], Dbias2_1[0], eb2[0]]).reshape(1, 3)
  W12c = jnp.pad(W12, ((0, 0), (0, 88)))
  W22c = jnp.pad(W22, ((0, 0), (64, 24)))

  hw, gates = _combine1(
      x, xw1, xw2, p_ei1, p_kf1, p_ei2, p_ks2, b11.reshape(1, 128),
      b21.reshape(1, 128), Gx, bx, Gh1, bh1, Gh2, bh2, W12c, W22c)

  q_a = spmm_e(hw, idx_e, w_e)
  spmm_p = _make_spmm_pair(128, ekpad, ekpad, 4)
  q_k = spmm_p(hw, idx_f, w_f, idx_s, w_s)

  b12p = jnp.pad(b12, (0, 8)).reshape(1, 48)
  b22p = jnp.pad(b22, (0, 8)).reshape(1, 48)
  out = _final(hw, gates, q_a, q_k, b12p, b22p)
  return out[:, :40]


# speedup vs baseline: 1.9896x; 1.9896x over previous
import functools

import jax
import jax.numpy as jnp
from jax import lax
from jax.experimental import pallas as pl
from jax.experimental.pallas import tpu as pltpu
from jax.experimental.pallas import tpu_sc as plsc

_N = 10000
_NPAD = 10016
_CH = 128
_NW = 32
_GAMMA = 0.1
_R = 400


def _spmm_body_maker(D, cpws, masks):
  nvec = D // 16

  def body_fn(table, edge_refs, out, src_v, dst_v, w_v, rows_v, acc, isems,
              gsem, c, s):
    wid = c * 16 + s

    def zrow(i, carry):
      for k in range(nvec):
        rows_v[i, pl.ds(k * 16, 16)] = jnp.zeros((16,), jnp.float32)
      return carry
    lax.fori_loop(0, _CH, zrow, 0)
    @pl.when(s < 15)
    def _zero():
      for j in range(5):
        pltpu.sync_copy(rows_v, acc.at[pl.ds(s * 640 + j * _CH, _CH)])
    @pl.when(s == 15)
    def _zero15():
      for j in range(3):
        pltpu.sync_copy(rows_v, acc.at[pl.ds(9600 + j * _CH, _CH)])
      pltpu.sync_copy(rows_v.at[pl.ds(0, 32)], acc.at[pl.ds(9984, 32)])
    plsc.subcore_barrier()

    for (srcr, dstr, wr), cpw, mask in zip(edge_refs, cpws, masks):
      e0 = wid * cpw * _CH
      def body(j, carry):
        off = e0 + j * _CH
        pltpu.sync_copy(srcr.at[pl.ds(off, _CH)], src_v.at[0])
        pltpu.sync_copy(dstr.at[pl.ds(off, _CH)], dst_v.at[0])
        pltpu.sync_copy(wr.at[pl.ds(off, _CH)], w_v.at[0])
        g_ = pltpu.make_async_copy(table.at[src_v.at[0]], rows_v, gsem)
        g_.start()
        g_.wait()
        def scale(g, c2):
          wvec = w_v[0, pl.ds(g * 16, 16)]
          for jj in range(16):
            wj = jnp.full((16,), wvec[jj], jnp.float32)
            i = g * 16 + jj
            for k in range(nvec):
              if k in mask:
                rows_v[i, pl.ds(k * 16, 16)] = (
                    rows_v[i, pl.ds(k * 16, 16)] * wj)
              else:
                rows_v[i, pl.ds(k * 16, 16)] = jnp.zeros((16,), jnp.float32)
          return c2
        lax.fori_loop(0, _CH // 16, scale, 0)
        pltpu.sync_copy(rows_v, acc.at[dst_v.at[0]], add=True)
        return carry
      lax.fori_loop(0, cpw, body, 0)

    plsc.subcore_barrier()
    r0 = s * 624
    for j in range(4):
      pltpu.sync_copy(acc.at[pl.ds(r0 + j * _CH, _CH)],
                      out.at[c, pl.ds(r0 + j * _CH, _CH)])
    pltpu.sync_copy(acc.at[pl.ds(r0 + 512, 112)],
                    out.at[c, pl.ds(r0 + 512, 112)])
    @pl.when(s == 15)
    def _tail():
      pltpu.sync_copy(acc.at[pl.ds(9984, 16)], out.at[c, pl.ds(9984, 16)])

  return body_fn


def _spmm_scratch(D):
  return [
      pltpu.VMEM((2, _CH), jnp.int32),
      pltpu.VMEM((2, _CH), jnp.int32),
      pltpu.VMEM((2, _CH), jnp.float32),
      pltpu.VMEM((_CH, D), jnp.float32),
      pltpu.VMEM_SHARED((_NPAD, D), jnp.float32),
      pltpu.SemaphoreType.DMA,
      pltpu.SemaphoreType.DMA,
      pltpu.SemaphoreType.DMA,
  ]


@functools.lru_cache(maxsize=None)
def _mesh():
  return plsc.VectorSubcoreMesh(core_axis_name="c", subcore_axis_name="s")


@functools.lru_cache(maxsize=None)
def _make_spmm(D, Epad):
  cpw = Epad // (_NW * _CH)
  body_fn = _spmm_body_maker(D, (cpw,), (tuple(range(D // 16)),))

  @functools.partial(
      pl.kernel,
      out_type=jax.ShapeDtypeStruct((2, _N, D), jnp.float32),
      mesh=_mesh(),
      scratch_types=_spmm_scratch(D),
  )
  def spmm(table, src, dst, w, out, src_v, dst_v, w_v, rows_v, acc,
           isem0, isem1, gsem):
    c = lax.axis_index("c")
    s = lax.axis_index("s")
    body_fn(table, ((src, dst, w),), out, src_v, dst_v, w_v, rows_v, acc,
            (isem0, isem1), gsem, c, s)

  return spmm


@functools.lru_cache(maxsize=None)
def _make_spmm_pair(D, Epad1, Epad2, halfvecs):
  cpw1 = Epad1 // (_NW * _CH)
  cpw2 = Epad2 // (_NW * _CH)
  body_fn = _spmm_body_maker(
      D, (cpw1, cpw2),
      (tuple(range(halfvecs)), tuple(range(halfvecs, 2 * halfvecs))))

  @functools.partial(
      pl.kernel,
      out_type=jax.ShapeDtypeStruct((2, _N, D), jnp.float32),
      mesh=_mesh(),
      scratch_types=_spmm_scratch(D),
  )
  def spmm(table, src1, dst1, w1, src2, dst2, w2, out,
           src_v, dst_v, w_v, rows_v, acc, isem0, isem1, gsem):
    c = lax.axis_index("c")
    s = lax.axis_index("s")
    body_fn(table, ((src1, dst1, w1), (src2, dst2, w2)), out,
            src_v, dst_v, w_v, rows_v, acc, (isem0, isem1), gsem, c, s)

  return spmm


def _mm2(x, Wa, Wb):
  def body(x_ref, wa_ref, wb_ref, oa_ref, ob_ref):
    xb = x_ref[...]
    oa_ref[...] = jnp.dot(xb, wa_ref[...], preferred_element_type=jnp.float32)
    ob_ref[...] = jnp.dot(xb, wb_ref[...], preferred_element_type=jnp.float32)
  return pl.pallas_call(
      body,
      grid=(_N // _R,),
      in_specs=[
          pl.BlockSpec((_R, 128), lambda i: (i, 0)),
          pl.BlockSpec((128, 128), lambda i: (0, 0)),
          pl.BlockSpec((128, 128), lambda i: (0, 0)),
      ],
      out_specs=[pl.BlockSpec((_R, 128), lambda i: (i, 0))] * 2,
      out_shape=[jax.ShapeDtypeStruct((_N, 128), jnp.float32)] * 2,
  )(x, Wa, Wb)


def _combine1(x, xw1, xw2, p_ei1, p_kf1, p_ei2, p_ks2, b11, b21,
              Gx, bx, Gh1, bh1, Gh2, bh2, W12p, W22p):
  def body(x_ref, xw1_ref, xw2_ref, pe1_ref, pk1_ref, pe2_ref, pk2_ref,
           b11_ref, b21_ref, gx_ref, bx_ref, gh1_ref, bh1_ref, gh2_ref,
           bh2_ref, w12_ref, w22_ref, hw_ref, gates_ref):
    xb = x_ref[...]
    g = jnp.dot(xb, gx_ref[...], preferred_element_type=jnp.float32) + bx_ref[...]
    s1 = jax.nn.sigmoid(g[:, 0:1])
    dk1 = g[:, 1:2]
    s2 = jax.nn.sigmoid(g[:, 2:3])
    dk2 = g[:, 3:4]
    a1 = pe1_ref[0] + pe1_ref[1] + b11_ref[...]
    k1 = pk1_ref[0] + pk1_ref[1] + b11_ref[...]
    i1 = xw1_ref[...] + b11_ref[...]
    h1 = jnp.maximum(s1 * a1 + (1.0 - s1) * k1 + _GAMMA * dk1 * i1, 0.0)
    a2 = pe2_ref[0] + pe2_ref[1] + b21_ref[...]
    k2 = pk2_ref[0] + pk2_ref[1] + b21_ref[...]
    i2 = xw2_ref[...] + b21_ref[...]
    h2 = jnp.maximum(s2 * a2 + (1.0 - s2) * k2 + _GAMMA * dk2 * i2, 0.0)
    hw_ref[...] = (
        jnp.dot(h1, w12_ref[...], preferred_element_type=jnp.float32)
        + jnp.dot(h2, w22_ref[...], preferred_element_type=jnp.float32))
    g1 = jnp.dot(h1, gh1_ref[...], preferred_element_type=jnp.float32) + bh1_ref[...]
    g2 = jnp.dot(h2, gh2_ref[...], preferred_element_type=jnp.float32) + bh2_ref[...]
    gates_ref[...] = jnp.concatenate([
        jax.nn.sigmoid(g1[:, 0:1]), g1[:, 1:2], jax.nn.sigmoid(g1[:, 2:3]),
        jax.nn.sigmoid(g2[:, 0:1]), g2[:, 1:2], jax.nn.sigmoid(g2[:, 2:3]),
        jnp.zeros_like(g1[:, 0:2]),
    ], axis=1)

  part = lambda: pl.BlockSpec((2, _R, 128), lambda i: (0, i, 0))
  return pl.pallas_call(
      body,
      grid=(_N // _R,),
      in_specs=[
          pl.BlockSpec((_R, 128), lambda i: (i, 0)),
          pl.BlockSpec((_R, 128), lambda i: (i, 0)),
          pl.BlockSpec((_R, 128), lambda i: (i, 0)),
          part(), part(), part(), part(),
          pl.BlockSpec((1, 128), lambda i: (0, 0)),
          pl.BlockSpec((1, 128), lambda i: (0, 0)),
          pl.BlockSpec((128, 4), lambda i: (0, 0)),
          pl.BlockSpec((1, 4), lambda i: (0, 0)),
          pl.BlockSpec((128, 3), lambda i: (0, 0)),
          pl.BlockSpec((1, 3), lambda i: (0, 0)),
          pl.BlockSpec((128, 3), lambda i: (0, 0)),
          pl.BlockSpec((1, 3), lambda i: (0, 0)),
          pl.BlockSpec((128, 128), lambda i: (0, 0)),
          pl.BlockSpec((128, 128), lambda i: (0, 0)),
      ],
      out_specs=[
          pl.BlockSpec((_R, 128), lambda i: (i, 0)),
          pl.BlockSpec((_R, 8), lambda i: (i, 0)),
      ],
      out_shape=[
          jax.ShapeDtypeStruct((_N, 128), jnp.float32),
          jax.ShapeDtypeStruct((_N, 8), jnp.float32),
      ],
  )(x, xw1, xw2, p_ei1, p_kf1, p_ei2, p_ks2, b11, b21,
    Gx, bx, Gh1, bh1, Gh2, bh2, W12p, W22p)


def _final(hw, gates, q_a, q_k, b12p, b22p):
  def body(hw_ref, g_ref, qa_ref, qk_ref, b12_ref, b22_ref, o_ref):
    g = g_ref[...]
    s21 = g[:, 0:1]
    dk21 = g[:, 1:2]
    w1 = g[:, 2:3]
    s22 = g[:, 3:4]
    dk22 = g[:, 4:5]
    w2 = g[:, 5:6]
    qa = qa_ref[0] + qa_ref[1]
    qk = qk_ref[0] + qk_ref[1]
    hw = hw_ref[...]
    a1 = qa[:, 0:48] + b12_ref[...]
    k1 = qk[:, 0:48] + b12_ref[...]
    i1 = hw[:, 0:48] + b12_ref[...]
    o1 = s21 * a1 + (1.0 - s21) * k1 + _GAMMA * dk21 * i1
    a2 = qa[:, 64:112] + b22_ref[...]
    k2 = qk[:, 64:112] + b22_ref[...]
    i2 = hw[:, 64:112] + b22_ref[...]
    o2 = s22 * a2 + (1.0 - s22) * k2 + _GAMMA * dk22 * i2
    out = w1 * o1 + w2 * o2
    ids = lax.broadcasted_iota(jnp.int32, out.shape, 1)
    valid = ids < 40
    m = jnp.max(jnp.where(valid, out, -jnp.inf), axis=1, keepdims=True)
    e = jnp.where(valid, jnp.exp(out - m), 0.0)
    se = jnp.sum(e, axis=1, keepdims=True)
    o_ref[...] = out - m - jnp.log(se)

  part = lambda: pl.BlockSpec((2, _R, 128), lambda i: (0, i, 0))
  return pl.pallas_call(
      body,
      grid=(_N // _R,),
      in_specs=[
          pl.BlockSpec((_R, 128), lambda i: (i, 0)),
          pl.BlockSpec((_R, 8), lambda i: (i, 0)),
          part(), part(),
          pl.BlockSpec((1, 48), lambda i: (0, 0)),
          pl.BlockSpec((1, 48), lambda i: (0, 0)),
      ],
      out_specs=pl.BlockSpec((_R, 48), lambda i: (i, 0)),
      out_shape=jax.ShapeDtypeStruct((_N, 48), jnp.float32),
  )(hw, gates, q_a, q_k, b12p, b22p)


def _pad_edges(ei, ew, epad):
  e = ew.shape[0]
  pad = epad - e
  src = jnp.concatenate([ei[0], jnp.zeros((pad,), jnp.int32)])
  dst = jnp.concatenate([ei[1], jnp.zeros((pad,), jnp.int32)])
  w = jnp.concatenate([ew, jnp.zeros((pad,), jnp.float32)])
  return src, dst, w


def kernel(x, edge_index, edge_weight, kf_edge_index, kf_edge_weight,
           ks_edge_index, ks_edge_weight, W11, b11, W12, b12, W21, b21,
           W22, b22, scores1_0, scores1_1, scores2_0, scores2_1, bias1_0,
           bias2_0, Dk1_0, Dk1_1, Dk2_0, Dk2_1, Dbias1_0, Dbias1_1,
           Dbias2_0, Dbias2_1, ec1, eb1, ec2, eb2):
  grain = _NW * _CH
  epad = ((edge_weight.shape[0] + grain - 1) // grain) * grain
  ekpad = ((kf_edge_weight.shape[0] + grain - 1) // grain) * grain
  src_e, dst_e, w_e = _pad_edges(edge_index, edge_weight, epad)
  src_f, dst_f, w_f = _pad_edges(kf_edge_index, kf_edge_weight, ekpad)
  src_s, dst_s, w_s = _pad_edges(ks_edge_index, ks_edge_weight, ekpad)

  xw1, xw2 = _mm2(x, W11, W21)

  spmm_e = _make_spmm(128, epad)
  spmm_k = _make_spmm(128, ekpad)
  p_ei1 = spmm_e(xw1, src_e, dst_e, w_e)
  p_kf1 = spmm_k(xw1, src_f, dst_f, w_f)
  p_ei2 = spmm_e(xw2, src_e, dst_e, w_e)
  p_ks2 = spmm_k(xw2, src_s, dst_s, w_s)

  Gx = jnp.concatenate([scores1_0, Dk1_0, scores2_0, Dk2_0], axis=1)
  bx = jnp.stack([bias1_0[0], Dbias1_0[0], bias2_0[0], Dbias2_0[0]]).reshape(1, 4)
  Gh1 = jnp.concatenate([scores1_1, Dk1_1, ec1], axis=1)
  bh1 = jnp.stack([bias1_0[0], Dbias1_1[0], eb1[0]]).reshape(1, 3)
  Gh2 = jnp.concatenate([scores2_1, Dk2_1, ec2], axis=1)
  bh2 = jnp.stack([bias2_0[0], Dbias2_1[0], eb2[0]]).reshape(1, 3)
  W12c = jnp.pad(W12, ((0, 0), (0, 88)))
  W22c = jnp.pad(W22, ((0, 0), (64, 24)))

  hw, gates = _combine1(
      x, xw1, xw2, p_ei1, p_kf1, p_ei2, p_ks2, b11.reshape(1, 128),
      b21.reshape(1, 128), Gx, bx, Gh1, bh1, Gh2, bh2, W12c, W22c)

  q_a = spmm_e(hw, src_e, dst_e, w_e)
  spmm_p = _make_spmm_pair(128, ekpad, ekpad, 4)
  q_k = spmm_p(hw, src_f, dst_f, w_f, src_s, dst_s, w_s)

  b12p = jnp.pad(b12, (0, 8)).reshape(1, 48)
  b22p = jnp.pad(b22, (0, 8)).reshape(1, 48)
  out = _final(hw, gates, q_a, q_k, b12p, b22p)
  return out[:, :40]
